# baseline (speedup 1.0000x reference)
import jax
import jax.numpy as jnp
from jax import lax
from jax.experimental import pallas as pl
from jax.experimental.pallas import tpu as pltpu
from jax.experimental.pallas import tpu_sc as plsc

_P = 40
_GW = 48
_NW = 32


def _agg_body(x_hbm, nbr_hbm, geo_hbm, out_hbm,
              idx_all, gv0, gv1, wv0, wv1, rows0, rows1, agg0, agg1,
              sg0, sg1, ss0, ss1):
    NB = x_hbm.shape[0]
    C = x_hbm.shape[2]
    nchunk = nbr_hbm.shape[1]
    per_worker = nchunk // _NW
    pairs = per_worker // 2
    wid = lax.axis_index("s") * 2 + lax.axis_index("c")
    base = wid * per_worker

    def cid_of(i):
        return base + i

    stage = [pltpu.make_async_copy(
        nbr_hbm.at[g, pl.ds(base, per_worker)],
        idx_all.at[pl.ds(g * per_worker, per_worker)], sg0)
        for g in range(4)]
    for c in stage:
        c.start()
    for c in stage:
        c.wait()

    def gather_copies(jj, gvb, rowsb, semg):
        copies = []
        for g in range(4):
            irow = idx_all.at[g * per_worker + jj]
            copies.append(pltpu.make_async_copy(
                geo_hbm.at[irow], gvb.at[pl.ds(g * _GW, _P)], semg))
            for b in range(NB):
                copies.append(pltpu.make_async_copy(
                    x_hbm.at[b].at[irow],
                    rowsb.at[b, pl.ds(g * _P, _P)], semg))
        return copies

    def issue(jj, gvb, rowsb, semg):
        for c in gather_copies(jj, gvb, rowsb, semg):
            c.start()

    def wait_gathers(jj, gvb, rowsb, semg):
        for c in gather_copies(jj, gvb, rowsb, semg):
            c.wait()

    def compute(gvb, wvb, rowsb, aggb):
        for grp in range(3):
            a = [gvb[pl.ds(k * _GW + grp * 16, 16)] for k in range(4)]
            m = jnp.maximum(jnp.maximum(a[0], a[1]), jnp.maximum(a[2], a[3]))
            e = [jnp.exp(ak - m) for ak in a]
            r = 1.0 / (e[0] + e[1] + e[2] + e[3])
            for k in range(4):
                wvb[pl.ds(k * _GW + grp * 16, 16)] = e[k] * r

        def point_body(p, carry2):
            w = [wvb[pl.ds(k * _GW + p, 16)][0] for k in range(4)]
            for b in range(NB):
                for c in range(C // 16):
                    sl = pl.ds(c * 16, 16)
                    acc = (w[0] * rowsb[b, p, sl]
                           + w[1] * rowsb[b, _P + p, sl]
                           + w[2] * rowsb[b, 2 * _P + p, sl]
                           + w[3] * rowsb[b, 3 * _P + p, sl])
                    aggb[b, p, sl] = acc
            return carry2

        lax.fori_loop(0, _P, point_body, 0)

    def store_copies(aggb, cid, sems):
        return [pltpu.make_async_copy(
            aggb.at[b], out_hbm.at[b, pl.ds(cid * _P, _P)], sems)
            for b in range(NB)]

    def store(aggb, cid, sems):
        for c in store_copies(aggb, cid, sems):
            c.start()

    def wait_store(aggb, cid, sems):
        for c in store_copies(aggb, cid, sems):
            c.wait()

    issue(0, gv0, rows0, sg0)

    def pair_body(t, carry):
        j0 = 2 * t
        j1 = 2 * t + 1
        j2 = jnp.minimum(2 * t + 2, per_worker - 1)
        issue(j1, gv1, rows1, sg1)
        wait_gathers(j0, gv0, rows0, sg0)

        @pl.when(t > 0)
        def _():
            wait_store(agg0, cid_of(j0), ss0)

        compute(gv0, wv0, rows0, agg0)
        store(agg0, cid_of(j0), ss0)
        issue(j2, gv0, rows0, sg0)
        wait_gathers(j1, gv1, rows1, sg1)

        @pl.when(t > 0)
        def _():
            wait_store(agg1, cid_of(j1), ss1)

        compute(gv1, wv1, rows1, agg1)
        store(agg1, cid_of(j1), ss1)
        return carry

    lax.fori_loop(0, pairs, pair_body, 0)
    wait_gathers(0, gv0, rows0, sg0)
    wait_store(agg0, cid_of(0), ss0)
    wait_store(agg1, cid_of(0), ss1)


def _aggregate(x_bnc, nbr3, geo):
    NB, N, C = x_bnc.shape
    mesh = plsc.VectorSubcoreMesh(core_axis_name="c", subcore_axis_name="s")
    nchunk = nbr3.shape[1]
    per_worker = nchunk // _NW
    f = pl.kernel(
        _agg_body,
        out_type=jax.ShapeDtypeStruct((NB, nchunk * _P, C), jnp.float32),
        mesh=mesh,
        scratch_types=(
            [pltpu.VMEM((4 * per_worker, _P), jnp.int32)]
            + [pltpu.VMEM((4 * _GW,), jnp.float32)] * 2
            + [pltpu.VMEM((4 * _GW + 16,), jnp.float32)] * 2
            + [pltpu.VMEM((NB, 4 * _P, C), jnp.float32)] * 2
            + [pltpu.VMEM((NB, _P, C), jnp.float32)] * 2
            + [pltpu.SemaphoreType.DMA] * 4
        ),
    )
    return f(x_bnc, nbr3, geo)


def _linear_body(a_ref, w_ref, b_ref, o_ref):
    a = a_ref[...]
    w = w_ref[...]
    bias = b_ref[...]
    NB, NT, CI = a.shape
    row_mod = lax.broadcasted_iota(jnp.int32, (NT, CI), 0) % 4
    masks = [row_mod == k for k in range(4)]
    wb = [w[k].astype(jnp.bfloat16) for k in range(4)]
    zero = jnp.bfloat16(0.0)
    outs = []
    for b in range(NB):
        ab = a[b].astype(jnp.bfloat16)
        acc = None
        for k in range(4):
            zk = jnp.where(masks[k], ab, zero)
            ok = lax.dot_general(
                zk, wb[k], (((1,), (0,)), ((), ())),
                preferred_element_type=jnp.float32)
            acc = ok if acc is None else acc + ok
        outs.append(acc + bias[None, :])
    o_ref[...] = jnp.stack(outs, axis=0)


def _linear(aggb, weight_r, bias, n_out):
    NB, _, CI = aggb.shape
    CO = weight_r.shape[2]
    NT = 4096
    return pl.pallas_call(
        _linear_body,
        grid=(pl.cdiv(n_out, NT),),
        in_specs=[
            pl.BlockSpec((NB, NT, CI), lambda q: (0, q, 0)),
            pl.BlockSpec((4, CI, CO), lambda q: (0, 0, 0)),
            pl.BlockSpec((CO,), lambda q: (0,)),
        ],
        out_specs=pl.BlockSpec((NB, NT, CO), lambda q: (0, q, 0)),
        out_shape=jax.ShapeDtypeStruct((NB, n_out, CO), jnp.float32),
    )(aggb, weight_r, bias)


def kernel(x, neighbors, weight, bias, geo_attention):
    B, C, N = x.shape
    x_bnc = jnp.transpose(x, (0, 2, 1))
    nchunk = N // _P
    nchunk_pad = ((nchunk + _NW - 1) // _NW) * _NW
    nbr3 = jnp.pad(neighbors.T, ((0, 0), (0, (nchunk_pad - nchunk) * _P))
                   ).reshape(4, nchunk_pad, _P)
    aggb = _aggregate(x_bnc, nbr3, geo_attention)
    weight_r = jnp.transpose(weight, (2, 1, 0))
    out_bnc = _linear(aggb, weight_r, bias, N)
    return jnp.transpose(out_bnc, (0, 2, 1))


# speedup vs baseline: 2.8051x; 2.8051x over previous
import jax
import jax.numpy as jnp
from jax import lax
from jax.experimental import pallas as pl
from jax.experimental.pallas import tpu as pltpu
from jax.experimental.pallas import tpu_sc as plsc

_P = 40
_GW = 48
_NW = 32


def _agg_body(x_hbm, nbr_hbm, geo_hbm, out_hbm,
              idx0, idx1, gv0, gv1, wv0, wv1, rows0, rows1, agg0, agg1,
              sg0, sg1, ss0, ss1):
    NB = x_hbm.shape[0]
    C = x_hbm.shape[2]
    N = x_hbm.shape[1]
    nchunk = N // _P
    per_worker = (nchunk + _NW - 1) // _NW
    pairs = per_worker // 2
    wid = lax.axis_index("s") * 2 + lax.axis_index("c")
    base = wid * per_worker

    def cid_of(i):
        return jnp.minimum(base + i, nchunk - 1)

    def idx_copies(cid, idxb, semg):
        return [pltpu.make_async_copy(
            nbr_hbm.at[pl.ds(g * N + cid * _P, _P)], idxb.at[g], semg)
            for g in range(4)]

    def gather_copies(cid, idxb, gvb, rowsb, semg):
        copies = []
        for g in range(4):
            copies.append(pltpu.make_async_copy(
                geo_hbm.at[idxb.at[g]], gvb.at[pl.ds(g * _GW, _P)], semg))
            for b in range(NB):
                copies.append(pltpu.make_async_copy(
                    x_hbm.at[b].at[idxb.at[g]],
                    rowsb.at[b, pl.ds(g * _P, _P)], semg))
        return copies

    def issue(cid, idxb, gvb, rowsb, semg):
        ics = idx_copies(cid, idxb, semg)
        for c in ics:
            c.start()
        for c in ics:
            c.wait()
        for c in gather_copies(cid, idxb, gvb, rowsb, semg):
            c.start()

    def wait_gathers(cid, idxb, gvb, rowsb, semg):
        for c in gather_copies(cid, idxb, gvb, rowsb, semg):
            c.wait()

    def compute(gvb, wvb, rowsb, aggb):
        for grp in range(3):
            a = [gvb[pl.ds(k * _GW + grp * 16, 16)] for k in range(4)]
            m = jnp.maximum(jnp.maximum(a[0], a[1]), jnp.maximum(a[2], a[3]))
            e = [jnp.exp(ak - m) for ak in a]
            r = 1.0 / (e[0] + e[1] + e[2] + e[3])
            for k in range(4):
                wvb[pl.ds(k * _GW + grp * 16, 16)] = e[k] * r

        def point_body(p, carry2):
            w = [wvb[pl.ds(k * _GW + p, 16)][0] for k in range(4)]
            for b in range(NB):
                for c in range(C // 16):
                    sl = pl.ds(c * 16, 16)
                    acc = (w[0] * rowsb[b, p, sl]
                           + w[1] * rowsb[b, _P + p, sl]
                           + w[2] * rowsb[b, 2 * _P + p, sl]
                           + w[3] * rowsb[b, 3 * _P + p, sl])
                    aggb[b, p, sl] = acc
            return carry2

        lax.fori_loop(0, _P, point_body, 0)

    def store_copies(aggb, cid, sems):
        return [pltpu.make_async_copy(
            aggb.at[b], out_hbm.at[b, pl.ds(cid * _P, _P)], sems)
            for b in range(NB)]

    def store(aggb, cid, sems):
        for c in store_copies(aggb, cid, sems):
            c.start()

    def wait_store(aggb, cid, sems):
        for c in store_copies(aggb, cid, sems):
            c.wait()

    issue(cid_of(0), idx0, gv0, rows0, sg0)

    def pair_body(t, carry):
        c0 = cid_of(2 * t)
        c1 = cid_of(2 * t + 1)
        c2 = cid_of(2 * t + 2)
        issue(c1, idx1, gv1, rows1, sg1)
        wait_gathers(c0, idx0, gv0, rows0, sg0)

        @pl.when(t > 0)
        def _():
            wait_store(agg0, c0, ss0)

        compute(gv0, wv0, rows0, agg0)
        store(agg0, c0, ss0)
        issue(c2, idx0, gv0, rows0, sg0)
        wait_gathers(c1, idx1, gv1, rows1, sg1)

        @pl.when(t > 0)
        def _():
            wait_store(agg1, c1, ss1)

        compute(gv1, wv1, rows1, agg1)
        store(agg1, c1, ss1)
        return carry

    lax.fori_loop(0, pairs, pair_body, 0)
    wait_gathers(cid_of(0), idx0, gv0, rows0, sg0)
    wait_store(agg0, cid_of(0), ss0)
    wait_store(agg1, cid_of(0), ss1)


def _aggregate(x_bnc, nbr_k, geo):
    NB, N, C = x_bnc.shape
    mesh = plsc.VectorSubcoreMesh(core_axis_name="c", subcore_axis_name="s")
    f = pl.kernel(
        _agg_body,
        out_type=jax.ShapeDtypeStruct((NB, N, C), jnp.float32),
        mesh=mesh,
        scratch_types=(
            [pltpu.VMEM((4, _P), jnp.int32)] * 2
            + [pltpu.VMEM((4 * _GW,), jnp.float32)] * 2
            + [pltpu.VMEM((4 * _GW + 16,), jnp.float32)] * 2
            + [pltpu.VMEM((NB, 4 * _P, C), jnp.float32)] * 2
            + [pltpu.VMEM((NB, _P, C), jnp.float32)] * 2
            + [pltpu.SemaphoreType.DMA] * 4
        ),
    )
    return f(x_bnc, nbr_k, geo)


def _linear_body(a_ref, w_ref, b_ref, o_ref):
    a = a_ref[...]
    w = w_ref[...]
    bias = b_ref[...]
    NB, NT, CI = a.shape
    row_mod = lax.broadcasted_iota(jnp.int32, (NT, CI), 0) % 4
    masks = [row_mod == k for k in range(4)]
    wb = [w[k].astype(jnp.bfloat16) for k in range(4)]
    zero = jnp.bfloat16(0.0)
    outs = []
    for b in range(NB):
        ab = a[b].astype(jnp.bfloat16)
        acc = None
        for k in range(4):
            zk = jnp.where(masks[k], ab, zero)
            ok = lax.dot_general(
                zk, wb[k], (((1,), (0,)), ((), ())),
                preferred_element_type=jnp.float32)
            acc = ok if acc is None else acc + ok
        outs.append(acc + bias[None, :])
    o_ref[...] = jnp.stack(outs, axis=0)


def _linear(aggb, weight_r, bias):
    NB, n_out, CI = aggb.shape
    CO = weight_r.shape[2]
    NT = 4096
    return pl.pallas_call(
        _linear_body,
        grid=(pl.cdiv(n_out, NT),),
        in_specs=[
            pl.BlockSpec((NB, NT, CI), lambda q: (0, q, 0)),
            pl.BlockSpec((4, CI, CO), lambda q: (0, 0, 0)),
            pl.BlockSpec((CO,), lambda q: (0,)),
        ],
        out_specs=pl.BlockSpec((NB, NT, CO), lambda q: (0, q, 0)),
        out_shape=jax.ShapeDtypeStruct((NB, n_out, CO), jnp.float32),
    )(aggb, weight_r, bias)


def kernel(x, neighbors, weight, bias, geo_attention):
    B, C, N = x.shape
    x_bnc = jnp.transpose(x, (0, 2, 1))
    nbr_k = neighbors.T.reshape(-1)
    aggb = _aggregate(x_bnc, nbr_k, geo_attention)
    weight_r = jnp.transpose(weight, (2, 1, 0))
    out_bnc = _linear(aggb, weight_r, bias)
    return jnp.transpose(out_bnc, (0, 2, 1))
